# baseline (speedup 1.0000x reference)
import jax
import jax.numpy as jnp
from jax.experimental import pallas as pl

_ROWS_PER_BLOCK = 128


def _copy_block(x_ref, o_ref):
    o_ref[...] = x_ref[...]


def kernel(x, idx, label):
    del idx, label
    rows, cols = x.shape
    grid = rows // _ROWS_PER_BLOCK
    return pl.pallas_call(
        _copy_block,
        out_shape=jax.ShapeDtypeStruct(x.shape, x.dtype),
        grid=(grid,),
        in_specs=[pl.BlockSpec((_ROWS_PER_BLOCK, cols), lambda i: (i, 0))],
        out_specs=pl.BlockSpec((_ROWS_PER_BLOCK, cols), lambda i: (i, 0)),
    )(x)
